# Initial kernel scaffold; baseline (speedup 1.0000x reference)
import jax
import jax.numpy as jnp
from jax.experimental import pallas as pl


def kernel(params, indices):
    raise NotImplementedError("write your pallas kernel here")


# speedup vs baseline: 1.7964x; 1.7964x over previous
import functools

import jax
import jax.numpy as jnp
from jax import lax
from jax.experimental import pallas as pl
from jax.experimental.pallas import tpu as pltpu
from jax.experimental.pallas import tpu_sc as plsc

_NC = 2
_NS = 16
_NW = _NC * _NS
_IDXW = 128


def _gather_body(n_chunks, k_rows, chunk, d,
                 table_hbm, idx_hbm, out_hbm, idx_v, rows_v, sem):
    wid = lax.axis_index("s") * _NC + lax.axis_index("c")
    row0 = wid * (n_chunks * k_rows)
    base = wid * (n_chunks * chunk)

    def body(g, carry):
        pltpu.sync_copy(idx_hbm.at[pl.ds(row0 + g * k_rows, k_rows)], idx_v)
        copies = [
            pltpu.async_copy(
                table_hbm.at[idx_v.at[j]],
                rows_v.at[pl.ds(j * _IDXW, _IDXW)],
                sem,
            )
            for j in range(k_rows)
        ]
        for c in copies:
            c.wait()
        pltpu.sync_copy(rows_v, out_hbm.at[pl.ds(base + g * chunk, chunk)])
        return carry

    lax.fori_loop(0, n_chunks, body, 0)


def kernel(params, indices):
    b, s = indices.shape
    v, d = params.shape
    n = b * s
    n_per_w = n // _NW
    chunk = 512
    k_rows = chunk // _IDXW
    n_chunks = n_per_w // chunk

    idx2d = indices.reshape(n // _IDXW, _IDXW).astype(jnp.int32)

    mesh = plsc.VectorSubcoreMesh(core_axis_name="c", subcore_axis_name="s")

    gather = functools.partial(
        pl.kernel,
        mesh=mesh,
        out_type=jax.ShapeDtypeStruct((n, d), jnp.float32),
        scratch_types=[
            pltpu.VMEM((k_rows, _IDXW), jnp.int32),
            pltpu.VMEM((chunk, d), jnp.float32),
            pltpu.SemaphoreType.DMA,
        ],
        compiler_params=pltpu.CompilerParams(use_tc_tiling_on_sc=False),
    )(functools.partial(_gather_body, n_chunks, k_rows, chunk, d))

    flat = gather(params, idx2d)
    return flat.reshape(b, s, d)


# speedup vs baseline: 1.8757x; 1.0441x over previous
import functools

import jax
import jax.numpy as jnp
from jax import lax
from jax.experimental import pallas as pl
from jax.experimental.pallas import tpu as pltpu
from jax.experimental.pallas import tpu_sc as plsc

_NC = 2
_NS = 16
_NW = _NC * _NS
_IDXW = 128
_CHUNK = 640
_K = _CHUNK // _IDXW


def _gather_body(n_chunks, rows_per_w,
                 table_hbm, idx_hbm, out_hbm,
                 idx_all, rows0, rows1, gsem0, gsem1, ssem0, ssem1):
    wid = lax.axis_index("s") * _NC + lax.axis_index("c")
    idx_row0 = wid * (rows_per_w // _IDXW)
    base = wid * rows_per_w
    rows = (rows0, rows1)
    gsems = (gsem0, gsem1)
    ssems = (ssem0, ssem1)

    pltpu.sync_copy(idx_hbm.at[pl.ds(idx_row0, rows_per_w // _IDXW)], idx_all)

    def fire_gathers(g, b):
        for j in range(_K):
            pltpu.async_copy(
                table_hbm.at[idx_all.at[g * _K + j]],
                rows[b].at[pl.ds(j * _IDXW, _IDXW)],
                gsems[b],
            )

    def wait_gathers(b):
        pltpu.make_async_copy(
            out_hbm.at[pl.ds(0, _CHUNK)], rows[b], gsems[b]
        ).wait()

    def fire_store(g, b):
        pltpu.async_copy(
            rows[b], out_hbm.at[pl.ds(base + g * _CHUNK, _CHUNK)], ssems[b]
        )

    def wait_store(b):
        pltpu.make_async_copy(
            rows[b], out_hbm.at[pl.ds(base, _CHUNK)], ssems[b]
        ).wait()

    fire_gathers(0, 0)
    fire_gathers(1, 1)
    wait_gathers(0)
    fire_store(0, 0)

    def pair(gg, carry):
        for b in (1, 0):
            g = 2 * gg + (1 if b == 1 else 2)
            wait_store(1 - b)
            fire_gathers(g + 1, 1 - b)
            wait_gathers(b)
            fire_store(g, b)
        return carry

    lax.fori_loop(0, (n_chunks - 2) // 2, pair, 0)

    bl = (n_chunks - 1) % 2
    wait_gathers(bl)
    fire_store(n_chunks - 1, bl)
    wait_store(1 - bl)
    wait_store(bl)


def kernel(params, indices):
    b, s = indices.shape
    v, d = params.shape
    n = b * s
    rows_per_w = n // _NW
    n_chunks = rows_per_w // _CHUNK

    idx2d = indices.reshape(n // _IDXW, _IDXW).astype(jnp.int32)

    mesh = plsc.VectorSubcoreMesh(core_axis_name="c", subcore_axis_name="s")

    gather = functools.partial(
        pl.kernel,
        mesh=mesh,
        out_type=jax.ShapeDtypeStruct((n, d), jnp.float32),
        scratch_types=[
            pltpu.VMEM((rows_per_w // _IDXW, _IDXW), jnp.int32),
            pltpu.VMEM((_CHUNK, d), jnp.float32),
            pltpu.VMEM((_CHUNK, d), jnp.float32),
            pltpu.SemaphoreType.DMA,
            pltpu.SemaphoreType.DMA,
            pltpu.SemaphoreType.DMA,
            pltpu.SemaphoreType.DMA,
        ],
        compiler_params=pltpu.CompilerParams(use_tc_tiling_on_sc=False),
    )(functools.partial(_gather_body, n_chunks, rows_per_w))

    flat = gather(params, idx2d)
    return flat.reshape(b, s, d)


# speedup vs baseline: 2.3232x; 1.2386x over previous
import functools

import jax
import jax.numpy as jnp
from jax import lax
from jax.experimental import pallas as pl
from jax.experimental.pallas import tpu as pltpu
from jax.experimental.pallas import tpu_sc as plsc

_NC = 2
_NS = 16
_NW = _NC * _NS
_CHI = 16
_PBLK = 4096


def _pack_body(in_ref, out_ref):
    t = in_ref[...].T
    y = t.reshape(_PBLK // 2, 2, 64)
    out_ref[...] = jnp.concatenate([y[:, 0, :], y[:, 1, :]], axis=1)


def _gather_body(n_chunks, rows_per_w, s, d,
                 table_hbm, idx_hbm, out_hbm,
                 idx_all, rows0, rows1, gsem0, gsem1, ssem0, ssem1):
    wid = lax.axis_index("s") * _NC + lax.axis_index("c")
    row0 = wid * rows_per_w
    rows = (rows0, rows1)
    gsems = (gsem0, gsem1)
    ssems = (ssem0, ssem1)

    pltpu.sync_copy(idx_hbm.at[pl.ds(row0, rows_per_w)], idx_all)

    def fire_gathers(g, b):
        for j in range(_CHI):
            pltpu.async_copy(
                table_hbm.at[idx_all.at[g * _CHI + j]],
                rows[b].at[j],
                gsems[b],
            )

    def wait_gathers(b):
        pltpu.make_async_copy(
            out_hbm.at[pl.ds(0, _CHI)], rows[b], gsems[b]
        ).wait()

    def fire_store(g, b):
        pltpu.async_copy(
            rows[b], out_hbm.at[pl.ds(row0 + g * _CHI, _CHI)], ssems[b]
        )

    def wait_store(b):
        pltpu.make_async_copy(
            rows[b], out_hbm.at[pl.ds(row0, _CHI)], ssems[b]
        ).wait()

    fire_gathers(0, 0)
    fire_gathers(1, 1)
    wait_gathers(0)
    fire_store(0, 0)

    def pair(gg, carry):
        for b in (1, 0):
            g = 2 * gg + (1 if b == 1 else 2)
            wait_store(1 - b)
            fire_gathers(g + 1, 1 - b)
            wait_gathers(b)
            fire_store(g, b)
        return carry

    lax.fori_loop(0, (n_chunks - 2) // 2, pair, 0)

    bl = (n_chunks - 1) % 2
    wait_gathers(bl)
    fire_store(n_chunks - 1, bl)
    wait_store(1 - bl)
    wait_store(bl)


def kernel(params, indices):
    b, s = indices.shape
    v, d = params.shape
    rows_per_w = b // _NW
    n_chunks = rows_per_w // _CHI

    idx32 = indices.astype(jnp.int32)

    pack = pl.pallas_call(
        _pack_body,
        grid=(pl.cdiv(v, _PBLK),),
        in_specs=[pl.BlockSpec((d, _PBLK), lambda i: (0, i))],
        out_specs=pl.BlockSpec((_PBLK // 2, 2 * d), lambda i: (i, 0)),
        out_shape=jax.ShapeDtypeStruct((v // 2, 2 * d), jnp.float32),
    )
    table_lin = pack(params.T).reshape(v, d)

    mesh = plsc.VectorSubcoreMesh(core_axis_name="c", subcore_axis_name="s")

    gather = functools.partial(
        pl.kernel,
        mesh=mesh,
        out_type=jax.ShapeDtypeStruct((b, s, d), jnp.float32),
        scratch_types=[
            pltpu.VMEM((rows_per_w, s), jnp.int32),
            pltpu.VMEM((_CHI, s, d), jnp.float32),
            pltpu.VMEM((_CHI, s, d), jnp.float32),
            pltpu.SemaphoreType.DMA,
            pltpu.SemaphoreType.DMA,
            pltpu.SemaphoreType.DMA,
            pltpu.SemaphoreType.DMA,
        ],
        compiler_params=pltpu.CompilerParams(use_tc_tiling_on_sc=False),
    )(functools.partial(_gather_body, n_chunks, rows_per_w, s, d))

    return gather(table_lin, idx32)


# speedup vs baseline: 4.5616x; 1.9635x over previous
import functools

import jax
import jax.numpy as jnp
from jax import lax
from jax.experimental import pallas as pl
from jax.experimental.pallas import tpu as pltpu
from jax.experimental.pallas import tpu_sc as plsc

_NC = 2
_NS = 16
_NW = _NC * _NS
_CHI = 16
_PBLK = 32768


def _pack_body(in_ref, out_ref):
    x = in_ref[...]
    tops = [x[:, c : c + 128] for c in range(0, _PBLK, 256)]
    bots = [x[:, c + 128 : c + 256] for c in range(0, _PBLK, 256)]
    z = jnp.concatenate(
        [jnp.concatenate(tops, axis=1), jnp.concatenate(bots, axis=1)], axis=0
    )
    out_ref[...] = z.T


def _gather_body(n_chunks, rows_per_w, s, d,
                 table_hbm, idx_hbm, out_hbm,
                 idx_all, rows0, rows1, gsem0, gsem1, ssem0, ssem1):
    wid = lax.axis_index("s") * _NC + lax.axis_index("c")
    row0 = wid * rows_per_w
    rows = (rows0, rows1)
    gsems = (gsem0, gsem1)
    ssems = (ssem0, ssem1)

    pltpu.sync_copy(idx_hbm.at[pl.ds(row0, rows_per_w)], idx_all)

    def fire_gathers(g, b):
        for j in range(_CHI):
            pltpu.async_copy(
                table_hbm.at[idx_all.at[g * _CHI + j]],
                rows[b].at[j],
                gsems[b],
            )

    def wait_gathers(b):
        pltpu.make_async_copy(
            out_hbm.at[pl.ds(0, _CHI)], rows[b], gsems[b]
        ).wait()

    def fire_store(g, b):
        pltpu.async_copy(
            rows[b],
            out_hbm.at[pl.ds(row0 + g * _CHI, _CHI), pl.ds(0, s), pl.ds(0, d)],
            ssems[b],
        )

    def wait_store(b):
        pltpu.make_async_copy(
            rows[b],
            out_hbm.at[pl.ds(row0, _CHI), pl.ds(0, s), pl.ds(0, d)],
            ssems[b],
        ).wait()

    fire_gathers(0, 0)
    fire_gathers(1, 1)
    wait_gathers(0)
    fire_store(0, 0)

    def pair(gg, carry):
        for b in (1, 0):
            g = 2 * gg + (1 if b == 1 else 2)
            wait_store(1 - b)
            fire_gathers(g + 1, 1 - b)
            wait_gathers(b)
            fire_store(g, b)
        return carry

    lax.fori_loop(0, (n_chunks - 2) // 2, pair, 0)

    bl = (n_chunks - 1) % 2
    wait_gathers(bl)
    fire_store(n_chunks - 1, bl)
    wait_store(1 - bl)
    wait_store(bl)


def kernel(params, indices):
    b, s = indices.shape
    v, d = params.shape
    rows_per_w = b // _NW
    n_chunks = rows_per_w // _CHI

    idx = indices.astype(jnp.int32)
    m = idx & 255
    idx32 = (idx - m) + 2 * (m & 127) + (m >> 7)

    vp = 2 * d * pl.cdiv(v, 4 * d)
    pack = pl.pallas_call(
        _pack_body,
        grid=(pl.cdiv(v, _PBLK),),
        in_specs=[pl.BlockSpec((d, _PBLK), lambda i: (0, i))],
        out_specs=pl.BlockSpec((_PBLK // 2, 2 * d), lambda i: (i, 0)),
        out_shape=jax.ShapeDtypeStruct((vp, 2 * d), jnp.float32),
    )
    table_lin = pack(params.T).reshape(2 * vp, d)

    mesh = plsc.VectorSubcoreMesh(core_axis_name="c", subcore_axis_name="s")

    gather = functools.partial(
        pl.kernel,
        mesh=mesh,
        out_type=jax.ShapeDtypeStruct((b, 56, 2 * d), jnp.float32),
        scratch_types=[
            pltpu.VMEM((rows_per_w, s), jnp.int32),
            pltpu.VMEM((_CHI, s, d), jnp.float32),
            pltpu.VMEM((_CHI, s, d), jnp.float32),
            pltpu.SemaphoreType.DMA,
            pltpu.SemaphoreType.DMA,
            pltpu.SemaphoreType.DMA,
            pltpu.SemaphoreType.DMA,
        ],
        compiler_params=pltpu.CompilerParams(use_tc_tiling_on_sc=False),
    )(functools.partial(_gather_body, n_chunks, rows_per_w, s, d))

    padded = gather(table_lin, idx32)
    return padded[:, :s, :d]
